# baseline (speedup 1.0000x reference)
import functools

import jax
import jax.numpy as jnp
from jax import lax
from jax.experimental import pallas as pl
from jax.experimental.pallas import tpu as pltpu
from jax.experimental.pallas import tpu_sc as plsc

N_NODES = 10000
D = 128
N_RADIAL = 2
N_THETA = 4
V = N_RADIAL * N_THETA
OUT = 64

NC, NS = 2, 16
NW = NC * NS
CHUNK_NODES = 8
CHUNK_ROWS = CHUNK_NODES * V
NGROUPS = CHUNK_ROWS // 16
FAST_CORE = 0
NCH_FAST = 64
NCH_SLOW = 16
FAST_NODES = NS * NCH_FAST * CHUNK_NODES
N_PAD = FAST_NODES + NS * NCH_SLOW * CHUNK_NODES


def _sc_interp(table, idx, w):
    mesh = plsc.VectorSubcoreMesh(core_axis_name="c", subcore_axis_name="s")

    @functools.partial(
        pl.kernel,
        mesh=mesh,
        out_type=jax.ShapeDtypeStruct((N_PAD, V * D), jnp.float32),
        scratch_types=[
            pltpu.VMEM((3, NCH_FAST, CHUNK_ROWS), jnp.int32),
            pltpu.VMEM((3, NCH_FAST, CHUNK_ROWS), jnp.float32),
            pltpu.VMEM((CHUNK_ROWS, D), jnp.float32),
            pltpu.VMEM((CHUNK_ROWS, D), jnp.float32),
            pltpu.VMEM((CHUNK_ROWS, D), jnp.float32),
            pltpu.VMEM((CHUNK_ROWS, D), jnp.float32),
            pltpu.VMEM((CHUNK_ROWS, D), jnp.float32),
            pltpu.VMEM((CHUNK_ROWS, D), jnp.float32),
            pltpu.VMEM((CHUNK_NODES, V * D), jnp.float32),
            pltpu.VMEM((CHUNK_NODES, V * D), jnp.float32),
            pltpu.SemaphoreType.DMA,
            pltpu.SemaphoreType.DMA,
            pltpu.SemaphoreType.DMA,
            pltpu.SemaphoreType.DMA,
        ],
    )
    def k(table_hbm, idx_hbm, w_hbm, out_hbm, idx_v, w_v,
          ga0, ga1, ga2, gb0, gb1, gb2, acca, accb,
          semga, semgb, semoa, semob):
        core = lax.axis_index("c")
        sub = lax.axis_index("s")
        is_fast = core == FAST_CORE
        nchunks = jnp.where(is_fast, NCH_FAST, NCH_SLOW)
        base = jnp.where(
            is_fast,
            sub * (NCH_FAST * CHUNK_NODES),
            FAST_NODES + sub * (NCH_SLOW * CHUNK_NODES))
        pltpu.sync_copy(idx_hbm.at[core, sub], idx_v)
        pltpu.sync_copy(w_hbm.at[core, sub], w_v)

        def fire_gathers(ci, g0, g1, g2, sem):
            pltpu.async_copy(table_hbm.at[idx_v.at[0, ci]], g0, sem)
            pltpu.async_copy(table_hbm.at[idx_v.at[1, ci]], g1, sem)
            pltpu.async_copy(table_hbm.at[idx_v.at[2, ci]], g2, sem)

        def wait_gathers(g0, g1, g2, sem):
            src = table_hbm.at[pl.ds(0, CHUNK_ROWS)]
            pltpu.make_async_copy(src, g0, sem).wait()
            pltpu.make_async_copy(src, g1, sem).wait()
            pltpu.make_async_copy(src, g2, sem).wait()

        def drain_out(acc, sem):
            pltpu.make_async_copy(
                acc, out_hbm.at[pl.ds(0, CHUNK_NODES)], sem).wait()

        def compute_chunk(ci, g0, g1, g2, acc):
            @plsc.parallel_loop(0, NGROUPS)
            def group_body(g):
                wv0 = w_v[0, ci, pl.ds(g * 16, 16)]
                wv1 = w_v[1, ci, pl.ds(g * 16, 16)]
                wv2 = w_v[2, ci, pl.ds(g * 16, 16)]
                for lane in range(16):
                    w0 = wv0[lane]
                    w1 = wv1[lane]
                    w2 = wv2[lane]
                    p = g * 16 + lane
                    nl = 2 * g + lane // V
                    f0 = (lane % V) * D
                    avs = [g0[p, pl.ds(q * 16, 16)] for q in range(D // 16)]
                    bvs = [g1[p, pl.ds(q * 16, 16)] for q in range(D // 16)]
                    cvs = [g2[p, pl.ds(q * 16, 16)] for q in range(D // 16)]
                    for q in range(D // 16):
                        acc[nl, pl.ds(f0 + q * 16, 16)] = (
                            avs[q] * w0 + bvs[q] * w1 + cvs[q] * w2)

        def do_chunk(ci, c2, g0, g1, g2, acc, semg, semo):
            wait_gathers(g0, g1, g2, semg)

            @pl.when(c2 >= 1)
            def _():
                drain_out(acc, semo)

            compute_chunk(ci, g0, g1, g2, acc)
            n0 = base + ci * CHUNK_NODES
            pltpu.async_copy(acc, out_hbm.at[pl.ds(n0, CHUNK_NODES)], semo)

            @pl.when(ci + 2 < nchunks)
            def _():
                fire_gathers(ci + 2, g0, g1, g2, semg)

        fire_gathers(0, ga0, ga1, ga2, semga)
        fire_gathers(1, gb0, gb1, gb2, semgb)

        def pair_body(c2, carry):
            do_chunk(2 * c2, c2, ga0, ga1, ga2, acca, semga, semoa)
            do_chunk(2 * c2 + 1, c2, gb0, gb1, gb2, accb, semgb, semob)
            return carry

        lax.fori_loop(0, nchunks // 2, pair_body, 0)
        drain_out(acca, semoa)
        drain_out(accb, semob)

    return k(table, idx, w)


def _tc_conv(pull2d, wstack):
    BM = 400

    def body(x_ref, w_ref, o_ref):
        x = x_ref[...]
        acc = jnp.dot(x, w_ref[0], preferred_element_type=jnp.float32)
        for r in range(1, N_THETA):
            acc = jnp.maximum(
                acc, jnp.dot(x, w_ref[r], preferred_element_type=jnp.float32))
        o_ref[...] = jnp.maximum(acc, 0.0)

    return pl.pallas_call(
        body,
        grid=(N_NODES // BM,),
        in_specs=[
            pl.BlockSpec((BM, V * D), lambda i: (i, 0)),
            pl.BlockSpec((N_THETA, V * D, OUT), lambda i: (0, 0, 0)),
        ],
        out_specs=pl.BlockSpec((BM, OUT), lambda i: (i, 0)),
        out_shape=jax.ShapeDtypeStruct((N_NODES, OUT), jnp.float32),
    )(pull2d, wstack)


def kernel(inputs, barycentric_coordinates, kernel):
    bc = barycentric_coordinates
    w = bc[..., 2::2]
    idx = bc[..., 3::2].astype(jnp.int32)
    pad = N_PAD - N_NODES

    def pack(a):
        a = jnp.pad(a, ((0, pad), (0, 0), (0, 0))).reshape(N_PAD * V, 3)
        fast = a[:FAST_NODES * V].reshape(NS, NCH_FAST, CHUNK_ROWS, 3)
        slow = a[FAST_NODES * V:].reshape(NS, NCH_SLOW, CHUNK_ROWS, 3)
        slow = jnp.pad(
            slow, ((0, 0), (0, NCH_FAST - NCH_SLOW), (0, 0), (0, 0)))
        cores = [fast, slow] if FAST_CORE == 0 else [slow, fast]
        return jnp.stack(cores, axis=0).transpose(0, 1, 4, 2, 3)

    pull = _sc_interp(inputs, pack(idx), pack(w).astype(jnp.float32))

    kf = kernel.reshape(V, OUT, D)
    rots = []
    for r in range(N_THETA):
        sl = [(vv // N_THETA) * N_THETA + ((vv % N_THETA) + r) % N_THETA
              for vv in range(V)]
        rots.append(kf[jnp.array(sl)].transpose(0, 2, 1).reshape(V * D, OUT))
    wstack = jnp.stack(rots, axis=0)
    return _tc_conv(pull, wstack)


# speedup vs baseline: 2.5160x; 2.5160x over previous
import functools

import jax
import jax.numpy as jnp
from jax import lax
from jax.experimental import pallas as pl
from jax.experimental.pallas import tpu as pltpu
from jax.experimental.pallas import tpu_sc as plsc

N_NODES = 10000
D = 128
N_RADIAL = 2
N_THETA = 4
V = N_RADIAL * N_THETA
OUT = 64

NC, NS = 2, 16
NW = NC * NS
CHUNK_NODES = 8
CHUNK_ROWS = CHUNK_NODES * V
NGROUPS = CHUNK_ROWS // 16
FAST_CORE = 0
NCH_FAST = 40
NCH_SLOW = 40
FAST_NODES = NS * NCH_FAST * CHUNK_NODES
N_PAD = FAST_NODES + NS * NCH_SLOW * CHUNK_NODES


def _sc_interp(table, idx, w):
    mesh = plsc.VectorSubcoreMesh(core_axis_name="c", subcore_axis_name="s")

    @functools.partial(
        pl.kernel,
        mesh=mesh,
        out_type=jax.ShapeDtypeStruct((N_PAD, V * D), jnp.float32),
        scratch_types=[
            pltpu.VMEM((3, NCH_FAST, CHUNK_ROWS), jnp.int32),
            pltpu.VMEM((3, NCH_FAST, CHUNK_ROWS), jnp.float32),
            pltpu.VMEM((CHUNK_ROWS, D), jnp.float32),
            pltpu.VMEM((CHUNK_ROWS, D), jnp.float32),
            pltpu.VMEM((CHUNK_ROWS, D), jnp.float32),
            pltpu.VMEM((CHUNK_ROWS, D), jnp.float32),
            pltpu.VMEM((CHUNK_ROWS, D), jnp.float32),
            pltpu.VMEM((CHUNK_ROWS, D), jnp.float32),
            pltpu.VMEM((CHUNK_NODES, V * D), jnp.float32),
            pltpu.VMEM((CHUNK_NODES, V * D), jnp.float32),
            pltpu.SemaphoreType.DMA,
            pltpu.SemaphoreType.DMA,
            pltpu.SemaphoreType.DMA,
            pltpu.SemaphoreType.DMA,
        ],
    )
    def k(table_hbm, idx_hbm, w_hbm, out_hbm, idx_v, w_v,
          ga0, ga1, ga2, gb0, gb1, gb2, acca, accb,
          semga, semgb, semoa, semob):
        core = lax.axis_index("c")
        sub = lax.axis_index("s")
        is_fast = core == FAST_CORE
        nchunks = jnp.where(is_fast, NCH_FAST, NCH_SLOW)
        base = jnp.where(
            is_fast,
            sub * (NCH_FAST * CHUNK_NODES),
            FAST_NODES + sub * (NCH_SLOW * CHUNK_NODES))
        pltpu.sync_copy(idx_hbm.at[core, sub], idx_v)
        pltpu.sync_copy(w_hbm.at[core, sub], w_v)

        def fire_gathers(ci, g0, g1, g2, sem):
            pltpu.async_copy(table_hbm.at[idx_v.at[0, ci]], g0, sem)
            pltpu.async_copy(table_hbm.at[idx_v.at[1, ci]], g1, sem)
            pltpu.async_copy(table_hbm.at[idx_v.at[2, ci]], g2, sem)

        def wait_gathers(g0, g1, g2, sem):
            src = table_hbm.at[pl.ds(0, CHUNK_ROWS)]
            pltpu.make_async_copy(src, g0, sem).wait()
            pltpu.make_async_copy(src, g1, sem).wait()
            pltpu.make_async_copy(src, g2, sem).wait()

        def drain_out(acc, sem):
            pltpu.make_async_copy(
                acc, out_hbm.at[pl.ds(0, CHUNK_NODES)], sem).wait()

        def compute_chunk(ci, g0, g1, g2, acc):
            @plsc.parallel_loop(0, NGROUPS)
            def group_body(g):
                wv0 = w_v[0, ci, pl.ds(g * 16, 16)]
                wv1 = w_v[1, ci, pl.ds(g * 16, 16)]
                wv2 = w_v[2, ci, pl.ds(g * 16, 16)]
                for lane in range(16):
                    w0 = wv0[lane]
                    w1 = wv1[lane]
                    w2 = wv2[lane]
                    p = g * 16 + lane
                    nl = 2 * g + lane // V
                    f0 = (lane % V) * D
                    avs = [g0[p, pl.ds(q * 16, 16)] for q in range(D // 16)]
                    bvs = [g1[p, pl.ds(q * 16, 16)] for q in range(D // 16)]
                    cvs = [g2[p, pl.ds(q * 16, 16)] for q in range(D // 16)]
                    for q in range(D // 16):
                        acc[nl, pl.ds(f0 + q * 16, 16)] = (
                            avs[q] * w0 + bvs[q] * w1 + cvs[q] * w2)

        def do_chunk(ci, c2, g0, g1, g2, acc, semg, semo):
            wait_gathers(g0, g1, g2, semg)

            @pl.when(c2 >= 1)
            def _():
                drain_out(acc, semo)

            compute_chunk(ci, g0, g1, g2, acc)
            n0 = base + ci * CHUNK_NODES
            pltpu.async_copy(acc, out_hbm.at[pl.ds(n0, CHUNK_NODES)], semo)

            @pl.when(ci + 2 < nchunks)
            def _():
                fire_gathers(ci + 2, g0, g1, g2, semg)

        fire_gathers(0, ga0, ga1, ga2, semga)
        fire_gathers(1, gb0, gb1, gb2, semgb)

        def pair_body(c2, carry):
            do_chunk(2 * c2, c2, ga0, ga1, ga2, acca, semga, semoa)
            do_chunk(2 * c2 + 1, c2, gb0, gb1, gb2, accb, semgb, semob)
            return carry

        lax.fori_loop(0, nchunks // 2, pair_body, 0)
        drain_out(acca, semoa)
        drain_out(accb, semob)

    return k(table, idx, w)


def _tc_conv(pull2d, wstack):
    BM = 400

    def body(x_ref, w_ref, o_ref):
        x = x_ref[...]
        acc = jnp.dot(x, w_ref[0], preferred_element_type=jnp.float32)
        for r in range(1, N_THETA):
            acc = jnp.maximum(
                acc, jnp.dot(x, w_ref[r], preferred_element_type=jnp.float32))
        o_ref[...] = jnp.maximum(acc, 0.0)

    return pl.pallas_call(
        body,
        grid=(N_NODES // BM,),
        in_specs=[
            pl.BlockSpec((BM, V * D), lambda i: (i, 0)),
            pl.BlockSpec((N_THETA, V * D, OUT), lambda i: (0, 0, 0)),
        ],
        out_specs=pl.BlockSpec((BM, OUT), lambda i: (i, 0)),
        out_shape=jax.ShapeDtypeStruct((N_NODES, OUT), jnp.float32),
    )(pull2d, wstack)


def kernel(inputs, barycentric_coordinates, kernel):
    bc = barycentric_coordinates
    w = bc[..., 2::2]
    idx = bc[..., 3::2].astype(jnp.int32)
    pad = N_PAD - N_NODES
    pad_idx = jnp.arange(pad * V * 3, dtype=jnp.int32).reshape(pad, V, 3)
    pad_idx = (pad_idx * 97) % N_NODES

    def pack(a):
        a = a.reshape(N_PAD * V, 3)
        fast = a[:FAST_NODES * V].reshape(NS, NCH_FAST, CHUNK_ROWS, 3)
        slow = a[FAST_NODES * V:].reshape(NS, NCH_SLOW, CHUNK_ROWS, 3)
        slow = jnp.pad(
            slow, ((0, 0), (0, NCH_FAST - NCH_SLOW), (0, 0), (0, 0)))
        cores = [fast, slow] if FAST_CORE == 0 else [slow, fast]
        return jnp.stack(cores, axis=0).transpose(0, 1, 4, 2, 3)

    ip = jnp.concatenate([idx, pad_idx], axis=0)
    wp = jnp.pad(w, ((0, pad), (0, 0), (0, 0)))
    pull = _sc_interp(inputs, pack(ip), pack(wp).astype(jnp.float32))

    kf = kernel.reshape(V, OUT, D)
    rots = []
    for r in range(N_THETA):
        sl = [(vv // N_THETA) * N_THETA + ((vv % N_THETA) + r) % N_THETA
              for vv in range(V)]
        rots.append(kf[jnp.array(sl)].transpose(0, 2, 1).reshape(V * D, OUT))
    wstack = jnp.stack(rots, axis=0)
    return _tc_conv(pull, wstack)


# speedup vs baseline: 3.0335x; 1.2057x over previous
import functools

import jax
import jax.numpy as jnp
from jax import lax
from jax.experimental import pallas as pl
from jax.experimental.pallas import tpu as pltpu
from jax.experimental.pallas import tpu_sc as plsc

N_NODES = 10000
D = 128
N_RADIAL = 2
N_THETA = 4
V = N_RADIAL * N_THETA
OUT = 64

NC, NS = 2, 16
NW = NC * NS
CHUNK_NODES = 8
CHUNK_ROWS = CHUNK_NODES * V
NGROUPS = CHUNK_ROWS // 16
FAST_CORE = 0
NCH_FAST = 40
NCH_SLOW = 40
FAST_NODES = NS * NCH_FAST * CHUNK_NODES
N_PAD = FAST_NODES + NS * NCH_SLOW * CHUNK_NODES


def _sc_interp(table, bct):
    mesh = plsc.VectorSubcoreMesh(core_axis_name="c", subcore_axis_name="s")
    rows_w = NCH_FAST * CHUNK_ROWS

    @functools.partial(
        pl.kernel,
        mesh=mesh,
        out_type=jax.ShapeDtypeStruct((N_PAD, V * D), jnp.float32),
        scratch_types=[
            pltpu.VMEM((3, NCH_FAST, CHUNK_ROWS), jnp.float32),
            pltpu.VMEM((3, NCH_FAST, CHUNK_ROWS), jnp.int32),
            pltpu.VMEM((3, NCH_FAST, CHUNK_ROWS), jnp.float32),
            pltpu.VMEM((CHUNK_ROWS, D), jnp.float32),
            pltpu.VMEM((CHUNK_ROWS, D), jnp.float32),
            pltpu.VMEM((CHUNK_ROWS, D), jnp.float32),
            pltpu.VMEM((CHUNK_ROWS, D), jnp.float32),
            pltpu.VMEM((CHUNK_ROWS, D), jnp.float32),
            pltpu.VMEM((CHUNK_ROWS, D), jnp.float32),
            pltpu.VMEM((CHUNK_NODES, V * D), jnp.float32),
            pltpu.VMEM((CHUNK_NODES, V * D), jnp.float32),
            pltpu.SemaphoreType.DMA,
            pltpu.SemaphoreType.DMA,
            pltpu.SemaphoreType.DMA,
            pltpu.SemaphoreType.DMA,
        ],
    )
    def k(table_hbm, bct_hbm, out_hbm, idxf_v, idx_v, w_v,
          ga0, ga1, ga2, gb0, gb1, gb2, acca, accb,
          semga, semgb, semoa, semob):
        core = lax.axis_index("c")
        sub = lax.axis_index("s")
        is_fast = core == FAST_CORE
        nchunks = jnp.where(is_fast, NCH_FAST, NCH_SLOW)
        base = jnp.where(
            is_fast,
            sub * (NCH_FAST * CHUNK_NODES),
            FAST_NODES + sub * (NCH_SLOW * CHUNK_NODES))
        nch_all = N_PAD * V // CHUNK_ROWS
        cbase = pl.multiple_of(base * V // CHUNK_ROWS, 8)
        for t in range(3):
            pltpu.sync_copy(
                bct_hbm.at[pl.ds((3 + 2 * t) * nch_all + cbase, NCH_FAST)],
                idxf_v.at[t])
            pltpu.sync_copy(
                bct_hbm.at[pl.ds((2 + 2 * t) * nch_all + cbase, NCH_FAST)],
                w_v.at[t])

        def conv_body(g, carry):
            ch = g // 4
            off = (g % 4) * 16
            for t in range(3):
                idx_v[t, ch, pl.ds(off, 16)] = (
                    idxf_v[t, ch, pl.ds(off, 16)].astype(jnp.int32))
            return carry

        lax.fori_loop(0, rows_w // 16, conv_body, 0)

        def fire_gathers(ci, g0, g1, g2, sem):
            pltpu.async_copy(table_hbm.at[idx_v.at[0, ci]], g0, sem)
            pltpu.async_copy(table_hbm.at[idx_v.at[1, ci]], g1, sem)
            pltpu.async_copy(table_hbm.at[idx_v.at[2, ci]], g2, sem)

        def wait_gathers(g0, g1, g2, sem):
            src = table_hbm.at[pl.ds(0, CHUNK_ROWS)]
            pltpu.make_async_copy(src, g0, sem).wait()
            pltpu.make_async_copy(src, g1, sem).wait()
            pltpu.make_async_copy(src, g2, sem).wait()

        def drain_out(acc, sem):
            pltpu.make_async_copy(
                acc, out_hbm.at[pl.ds(0, CHUNK_NODES)], sem).wait()

        def compute_chunk(ci, g0, g1, g2, acc):
            @plsc.parallel_loop(0, NGROUPS)
            def group_body(g):
                wv0 = w_v[0, ci, pl.ds(g * 16, 16)]
                wv1 = w_v[1, ci, pl.ds(g * 16, 16)]
                wv2 = w_v[2, ci, pl.ds(g * 16, 16)]
                for lane in range(16):
                    w0 = wv0[lane]
                    w1 = wv1[lane]
                    w2 = wv2[lane]
                    p = g * 16 + lane
                    nl = 2 * g + lane // V
                    f0 = (lane % V) * D
                    avs = [g0[p, pl.ds(q * 16, 16)] for q in range(D // 16)]
                    bvs = [g1[p, pl.ds(q * 16, 16)] for q in range(D // 16)]
                    cvs = [g2[p, pl.ds(q * 16, 16)] for q in range(D // 16)]
                    for q in range(D // 16):
                        acc[nl, pl.ds(f0 + q * 16, 16)] = (
                            avs[q] * w0 + bvs[q] * w1 + cvs[q] * w2)

        def do_chunk(ci, c2, g0, g1, g2, acc, semg, semo):
            wait_gathers(g0, g1, g2, semg)

            @pl.when(c2 >= 1)
            def _():
                drain_out(acc, semo)

            compute_chunk(ci, g0, g1, g2, acc)
            n0 = base + ci * CHUNK_NODES
            pltpu.async_copy(acc, out_hbm.at[pl.ds(n0, CHUNK_NODES)], semo)

            @pl.when(ci + 2 < nchunks)
            def _():
                fire_gathers(ci + 2, g0, g1, g2, semg)

        fire_gathers(0, ga0, ga1, ga2, semga)
        fire_gathers(1, gb0, gb1, gb2, semgb)

        def pair_body(c2, carry):
            do_chunk(2 * c2, c2, ga0, ga1, ga2, acca, semga, semoa)
            do_chunk(2 * c2 + 1, c2, gb0, gb1, gb2, accb, semgb, semob)
            return carry

        lax.fori_loop(0, nchunks // 2, pair_body, 0)
        drain_out(acca, semoa)
        drain_out(accb, semob)

    return k(table, bct)


def _tc_conv(pull2d, wstack):
    BM = 400

    def body(x_ref, w_ref, o_ref):
        x = x_ref[...]
        acc = jnp.dot(x, w_ref[0], preferred_element_type=jnp.float32)
        for r in range(1, N_THETA):
            acc = jnp.maximum(
                acc, jnp.dot(x, w_ref[r], preferred_element_type=jnp.float32))
        o_ref[...] = jnp.maximum(acc, 0.0)

    return pl.pallas_call(
        body,
        grid=(N_NODES // BM,),
        in_specs=[
            pl.BlockSpec((BM, V * D), lambda i: (i, 0)),
            pl.BlockSpec((N_THETA, V * D, OUT), lambda i: (0, 0, 0)),
        ],
        out_specs=pl.BlockSpec((BM, OUT), lambda i: (i, 0)),
        out_shape=jax.ShapeDtypeStruct((N_NODES, OUT), jnp.float32),
    )(pull2d, wstack)


def kernel(inputs, barycentric_coordinates, kernel):
    bc = barycentric_coordinates
    pad = N_PAD - N_NODES
    pad_idx = jnp.arange(pad * V * 3, dtype=jnp.int32).reshape(3, pad * V)
    pad_idx = ((pad_idx * 97) % N_NODES).astype(jnp.float32)
    pad_bct = jnp.zeros((8, pad * V), jnp.float32).at[3::2, :].set(pad_idx)
    bct = jnp.concatenate(
        [bc.reshape(N_NODES * V, 8).T, pad_bct], axis=1)
    bct = bct.reshape(8 * (N_PAD * V // CHUNK_ROWS), CHUNK_ROWS)
    pull = _sc_interp(inputs, bct)

    kf = kernel.reshape(V, OUT, D)
    rots = []
    for r in range(N_THETA):
        sl = [(vv // N_THETA) * N_THETA + ((vv % N_THETA) + r) % N_THETA
              for vv in range(V)]
        rots.append(kf[jnp.array(sl)].transpose(0, 2, 1).reshape(V * D, OUT))
    wstack = jnp.stack(rots, axis=0)
    return _tc_conv(pull, wstack)
